# Initial kernel scaffold; baseline (speedup 1.0000x reference)
import jax
import jax.numpy as jnp
from jax.experimental import pallas as pl


def kernel(x, edge_index, batch, W_proj, b_proj, Wq, bq, Wk, bk, Wv, bv, Wskip, bskip, Wbeta, gamma, beta_bn):
    raise NotImplementedError("write your pallas kernel here")


# speedup vs baseline: 21.4803x; 21.4803x over previous
import math

import jax
import jax.numpy as jnp
from jax import lax
from jax.experimental import pallas as pl
from jax.experimental.pallas import tpu as pltpu
from jax.experimental.pallas import tpu_sc as plsc

N = 10000
E = 320000
D_IN = 128
HID = 256
HEADS = 8
DH = 32
L = 4
G = 16

RB = 1000
NBLK = N // RB

NSC = 2
NTILE = 16
HPC = HEADS // NSC
EPT = E // NTILE
C = 64
NCHUNK = EPT // C
CT = EPT - NCHUNK * C
GPC = C // 16
NPAD = 10240
NROWS = NPAD // NTILE
DROWS = HPC * NPAD // NTILE
SCALE = 1.0 / math.sqrt(DH)

F32 = jnp.float32
I32 = jnp.int32


def _projin_body(x_ref, w_ref, b_ref, o_ref):
    o_ref[...] = jnp.dot(x_ref[...], w_ref[...],
                         preferred_element_type=F32) + b_ref[...]


def _head_mask():
    d = lax.broadcasted_iota(I32, (HID, HEADS), 0) // DH
    h = lax.broadcasted_iota(I32, (HID, HEADS), 1)
    return (d == h).astype(F32)


def _proj_core(h, wq, bq, wk, bk, wv, bv, ws, bs,
               qT_ref, kvT_ref, r_ref, bb_ref, mq_s, mk_s, i):
    q = (jnp.dot(h, wq, preferred_element_type=F32) + bq) * SCALE
    k = jnp.dot(h, wk, preferred_element_type=F32) + bk
    v = jnp.dot(h, wv, preferred_element_type=F32) + bv
    r_ref[...] = jnp.dot(h, ws, preferred_element_type=F32) + bs
    for cc in range(NSC):
        sl = slice(cc * HPC * DH, (cc + 1) * HPC * DH)
        qT_ref[cc] = q[:, sl]
        kvT_ref[cc, :, 0:HPC * DH] = k[:, sl]
        kvT_ref[cc, :, HPC * DH:2 * HPC * DH] = v[:, sl]
    hm = _head_mask()
    mq = jnp.max(jnp.dot(q * q, hm, preferred_element_type=F32),
                 axis=0, keepdims=True)
    mk = jnp.max(jnp.dot(k * k, hm, preferred_element_type=F32),
                 axis=0, keepdims=True)

    @pl.when(i == 0)
    def _():
        mq_s[...] = mq
        mk_s[...] = mk

    @pl.when(i > 0)
    def _():
        mq_s[...] = jnp.maximum(mq_s[...], mq)
        mk_s[...] = jnp.maximum(mk_s[...], mk)

    @pl.when(i == NBLK - 1)
    def _():
        bb_ref[...] = jnp.maximum(jnp.sqrt(mq_s[...] * mk_s[...]) - 30.0, 0.0)


def _proj0_body(h_ref, wq_ref, bq_ref, wk_ref, bk_ref, wv_ref, bv_ref,
                ws_ref, bs_ref, qT_ref, kvT_ref, r_ref, bb_ref, mq_s, mk_s):
    i = pl.program_id(0)
    _proj_core(h_ref[...], wq_ref[...], bq_ref[...], wk_ref[...], bk_ref[...],
               wv_ref[...], bv_ref[...], ws_ref[...], bs_ref[...],
               qT_ref, kvT_ref, r_ref, bb_ref, mq_s, mk_s, i)


def _projbn_body(h_ref, ssum_ref, sqsum_ref, gam_ref, bet_ref,
                 wq_ref, bq_ref, wk_ref, bk_ref, wv_ref, bv_ref,
                 ws_ref, bs_ref, qT_ref, kvT_ref, r_ref, bb_ref, mq_s, mk_s):
    i = pl.program_id(0)
    mu = ssum_ref[...] * (1.0 / N)
    var = sqsum_ref[...] * (1.0 / N) - mu * mu
    h = (h_ref[...] - mu) * lax.rsqrt(var + 1e-5) * gam_ref[...] + bet_ref[...]
    h = jnp.maximum(h, 0.0)
    _proj_core(h, wq_ref[...], bq_ref[...], wk_ref[...], bk_ref[...],
               wv_ref[...], bv_ref[...], ws_ref[...], bs_ref[...],
               qT_ref, kvT_ref, r_ref, bb_ref, mq_s, mk_s, i)


def _combine_body(num_ref, den_ref, r_ref, wb_ref, hpre_ref, ssum_ref,
                  sqsum_ref, acc_s, acc2_s):
    i = pl.program_id(0)
    r = r_ref[...]
    out = jnp.concatenate([num_ref[0], num_ref[1]], axis=1)
    out = out / (den_ref[...] + 1e-16)
    cat = jnp.concatenate([out, r, out - r], axis=1)
    g = jax.nn.sigmoid(jnp.dot(cat, wb_ref[...], preferred_element_type=F32))
    hn = g * r + (1.0 - g) * out
    hpre_ref[...] = hn
    ps = jnp.sum(hn, axis=0, keepdims=True)
    ps2 = jnp.sum(hn * hn, axis=0, keepdims=True)

    @pl.when(i == 0)
    def _():
        acc_s[...] = ps
        acc2_s[...] = ps2

    @pl.when(i > 0)
    def _():
        acc_s[...] += ps
        acc2_s[...] += ps2

    @pl.when(i == NBLK - 1)
    def _():
        ssum_ref[...] = acc_s[...]
        sqsum_ref[...] = acc2_s[...]


def _pool_body(h_ref, ssum_ref, sqsum_ref, gam_ref, bet_ref, bat_ref,
               out_ref, acc_s, cnt_s):
    i = pl.program_id(0)
    mu = ssum_ref[...] * (1.0 / N)
    var = sqsum_ref[...] * (1.0 / N) - mu * mu
    h = (h_ref[...] - mu) * lax.rsqrt(var + 1e-5) * gam_ref[...] + bet_ref[...]
    h = jnp.maximum(h, 0.0)
    b = bat_ref[0]
    gi = lax.broadcasted_iota(I32, (G, RB), 0)
    oh = (gi == jnp.broadcast_to(b, (G, RB))).astype(F32)
    ps = jnp.dot(oh, h, preferred_element_type=F32)
    pc = jnp.broadcast_to(jnp.sum(oh, axis=1, keepdims=True), (G, HID))

    @pl.when(i == 0)
    def _():
        acc_s[...] = ps
        cnt_s[...] = pc

    @pl.when(i > 0)
    def _():
        acc_s[...] += ps
        cnt_s[...] += pc

    @pl.when(i == NBLK - 1)
    def _():
        out_ref[...] = acc_s[...] / (cnt_s[...] + 1e-16)


_projin_call = pl.pallas_call(
    _projin_body,
    grid=(NBLK,),
    in_specs=[
        pl.BlockSpec((RB, D_IN), lambda i: (i, 0)),
        pl.BlockSpec((D_IN, HID), lambda i: (0, 0)),
        pl.BlockSpec((1, HID), lambda i: (0, 0)),
    ],
    out_specs=pl.BlockSpec((RB, HID), lambda i: (i, 0)),
    out_shape=jax.ShapeDtypeStruct((N, HID), F32),
)

_W_SPECS = [
    pl.BlockSpec((HID, HID), lambda i: (0, 0)),
    pl.BlockSpec((1, HID), lambda i: (0, 0)),
] * 4

_PROJ_OUT = dict(
    out_specs=[
        pl.BlockSpec((NSC, RB, HPC * DH), lambda i: (0, i, 0)),
        pl.BlockSpec((NSC, RB, 2 * HPC * DH), lambda i: (0, i, 0)),
        pl.BlockSpec((RB, HID), lambda i: (i, 0)),
        pl.BlockSpec((1, HEADS), lambda i: (0, 0)),
    ],
    out_shape=[
        jax.ShapeDtypeStruct((NSC, N, HPC * DH), F32),
        jax.ShapeDtypeStruct((NSC, N, 2 * HPC * DH), F32),
        jax.ShapeDtypeStruct((N, HID), F32),
        jax.ShapeDtypeStruct((1, HEADS), F32),
    ],
    scratch_shapes=[pltpu.VMEM((1, HEADS), F32), pltpu.VMEM((1, HEADS), F32)],
)

_proj0_call = pl.pallas_call(
    _proj0_body,
    grid=(NBLK,),
    in_specs=[pl.BlockSpec((RB, HID), lambda i: (i, 0))] + _W_SPECS,
    **_PROJ_OUT,
)

_projbn_call = pl.pallas_call(
    _projbn_body,
    grid=(NBLK,),
    in_specs=[
        pl.BlockSpec((RB, HID), lambda i: (i, 0)),
        pl.BlockSpec((1, HID), lambda i: (0, 0)),
        pl.BlockSpec((1, HID), lambda i: (0, 0)),
        pl.BlockSpec((1, HID), lambda i: (0, 0)),
        pl.BlockSpec((1, HID), lambda i: (0, 0)),
    ] + _W_SPECS,
    **_PROJ_OUT,
)

_combine_call = pl.pallas_call(
    _combine_body,
    grid=(NBLK,),
    in_specs=[
        pl.BlockSpec((NSC, RB, HPC * DH), lambda i: (0, i, 0)),
        pl.BlockSpec((RB, HID), lambda i: (i, 0)),
        pl.BlockSpec((RB, HID), lambda i: (i, 0)),
        pl.BlockSpec((3 * HID, 1), lambda i: (0, 0)),
    ],
    out_specs=[
        pl.BlockSpec((RB, HID), lambda i: (i, 0)),
        pl.BlockSpec((1, HID), lambda i: (0, 0)),
        pl.BlockSpec((1, HID), lambda i: (0, 0)),
    ],
    out_shape=[
        jax.ShapeDtypeStruct((N, HID), F32),
        jax.ShapeDtypeStruct((1, HID), F32),
        jax.ShapeDtypeStruct((1, HID), F32),
    ],
    scratch_shapes=[pltpu.VMEM((1, HID), F32), pltpu.VMEM((1, HID), F32)],
)

_pool_call = pl.pallas_call(
    _pool_body,
    grid=(NBLK,),
    in_specs=[
        pl.BlockSpec((RB, HID), lambda i: (i, 0)),
        pl.BlockSpec((1, HID), lambda i: (0, 0)),
        pl.BlockSpec((1, HID), lambda i: (0, 0)),
        pl.BlockSpec((1, HID), lambda i: (0, 0)),
        pl.BlockSpec((1, HID), lambda i: (0, 0)),
        pl.BlockSpec((1, 1, RB), lambda i: (i, 0, 0)),
    ],
    out_specs=pl.BlockSpec((G, HID), lambda i: (0, 0)),
    out_shape=jax.ShapeDtypeStruct((G, HID), F32),
    scratch_shapes=[pltpu.VMEM((G, HID), F32), pltpu.VMEM((G, HID), F32)],
)


def _edge_body(qT, kvT, srcg, dstg, bb, out_num, out_den,
               num_s, den_s, qbuf, kvbuf, ob, wbuf, zbuf,
               srcb, dstb, idxq, idxkv, idxoh,
               srcb_t, dstb_t, idxq_t, idxkv_t, idxoh_t,
               bbuf, semg, semo):
    c = lax.axis_index("c")
    s = lax.axis_index("s")
    iota16 = lax.iota(I32, 16)
    zero16 = jnp.zeros((16,), F32)
    cN = c * N

    pltpu.sync_copy(bb.at[c], bbuf)

    def _zrow(j, _):
        for kk in range(8):
            ob[j, pl.ds(kk * 16, 16)] = zero16
        return _

    lax.fori_loop(0, C, _zrow, None)
    for hh in range(HPC):
        for kk in range(C // 16):
            wbuf[hh, pl.ds(kk * 16, 16)] = zero16
    for kk in range(32):
        zbuf[pl.ds(kk * 16, 16)] = zero16
    for z in range(NROWS // C):
        pltpu.sync_copy(ob, num_s.at[pl.ds(s * NROWS + z * C, C)])
    for z in range(DROWS // 512):
        pltpu.sync_copy(zbuf, den_s.at[pl.ds(s * DROWS + z * 512, 512)])
    plsc.subcore_barrier()

    def _compute(n_e):
        ng = n_e // 16
        for hh in range(HPC):
            bv = bbuf[hh]
            h0 = hh * DH
            v0 = HPC * DH + hh * DH

            def _group(g, _):
                acc = zero16
                for jj in range(16):
                    j = g * 16 + jj
                    p = (qbuf[j, pl.ds(h0, 16)] * kvbuf[j, pl.ds(h0, 16)] +
                         qbuf[j, pl.ds(h0 + 16, 16)] *
                         kvbuf[j, pl.ds(h0 + 16, 16)])
                    sval = jnp.sum(p)
                    acc = jnp.where(iota16 == jj, jnp.full((16,), sval, F32),
                                    acc)
                w16 = jnp.exp(acc - bv)
                wbuf[hh, pl.ds(g * 16, 16)] = w16
                for jj in range(16):
                    j = g * 16 + jj
                    ws = w16[jj]
                    ob[j, pl.ds(h0, 16)] = ws * kvbuf[j, pl.ds(v0, 16)]
                    ob[j, pl.ds(h0 + 16, 16)] = ws * kvbuf[j, pl.ds(v0 + 16,
                                                                    16)]
                return _

            lax.fori_loop(0, ng, _group, None)

    def _chunk(t, _):
        base = s * EPT + t * C
        pltpu.sync_copy(srcg.at[pl.ds(base, C)], srcb)
        pltpu.sync_copy(dstg.at[pl.ds(base, C)], dstb)

        def _gidx(g, _):
            dv = dstb[pl.ds(g * 16, 16)]
            sv = srcb[pl.ds(g * 16, 16)]
            idxq[pl.ds(g * 16, 16)] = dv + cN
            idxkv[pl.ds(g * 16, 16)] = sv + cN
            for hh in range(HPC):
                idxoh[hh, pl.ds(g * 16, 16)] = dv + hh * NPAD
            return _

        lax.fori_loop(0, GPC, _gidx, None)

        cp1 = pltpu.async_copy(qT.at[idxq], qbuf, semg)
        cp2 = pltpu.async_copy(kvT.at[idxkv], kvbuf, semg)
        cp1.wait()
        cp2.wait()

        _compute(C)

        sps = [pltpu.async_copy(ob, num_s.at[dstb], semo, add=True)]
        for hh in range(HPC):
            sps.append(pltpu.async_copy(wbuf.at[hh],
                                        den_s.at[idxoh.at[hh]], semo,
                                        add=True))
        for cp in sps:
            cp.wait()
        return _

    lax.fori_loop(0, NCHUNK, _chunk, None)

    base_t = s * EPT + NCHUNK * C
    pltpu.sync_copy(srcg.at[pl.ds(base_t, CT)], srcb_t)
    pltpu.sync_copy(dstg.at[pl.ds(base_t, CT)], dstb_t)
    for g in range(CT // 16):
        dv = dstb_t[pl.ds(g * 16, 16)]
        sv = srcb_t[pl.ds(g * 16, 16)]
        idxq_t[pl.ds(g * 16, 16)] = dv + cN
        idxkv_t[pl.ds(g * 16, 16)] = sv + cN
        for hh in range(HPC):
            idxoh_t[hh, pl.ds(g * 16, 16)] = dv + hh * NPAD
    cp1 = pltpu.async_copy(qT.at[idxq_t], qbuf.at[pl.ds(0, CT)], semg)
    cp2 = pltpu.async_copy(kvT.at[idxkv_t], kvbuf.at[pl.ds(0, CT)], semg)
    cp1.wait()
    cp2.wait()
    _compute(CT)
    sps = [pltpu.async_copy(ob.at[pl.ds(0, CT)], num_s.at[dstb_t], semo,
                            add=True)]
    for hh in range(HPC):
        sps.append(pltpu.async_copy(wbuf.at[hh, pl.ds(0, CT)],
                                    den_s.at[idxoh_t.at[hh]], semo, add=True))
    for cp in sps:
        cp.wait()

    plsc.subcore_barrier()
    pltpu.sync_copy(num_s.at[pl.ds(s * NROWS, NROWS)],
                    out_num.at[pl.ds(c * NPAD + s * NROWS, NROWS)])
    pltpu.sync_copy(den_s.at[pl.ds(s * DROWS, DROWS)],
                    out_den.at[pl.ds(c * HPC * NPAD + s * DROWS, DROWS)])


_edge_call = pl.kernel(
    _edge_body,
    out_type=(jax.ShapeDtypeStruct((NSC * NPAD, 128), F32),
              jax.ShapeDtypeStruct((NSC * HPC * NPAD,), F32)),
    mesh=plsc.VectorSubcoreMesh(core_axis_name="c", subcore_axis_name="s",
                                num_cores=NSC, num_subcores=NTILE),
    compiler_params=pltpu.CompilerParams(needs_layout_passes=False),
    scratch_types=[
        pltpu.VMEM_SHARED((NPAD, 128), F32),
        pltpu.VMEM_SHARED((HPC * NPAD,), F32),
        pltpu.VMEM((C, 128), F32),
        pltpu.VMEM((C, 2 * 128), F32),
        pltpu.VMEM((C, 128), F32),
        pltpu.VMEM((HPC, C), F32),
        pltpu.VMEM((512,), F32),
        pltpu.VMEM((C,), I32),
        pltpu.VMEM((C,), I32),
        pltpu.VMEM((C,), I32),
        pltpu.VMEM((C,), I32),
        pltpu.VMEM((HPC, C), I32),
        pltpu.VMEM((CT,), I32),
        pltpu.VMEM((CT,), I32),
        pltpu.VMEM((CT,), I32),
        pltpu.VMEM((CT,), I32),
        pltpu.VMEM((HPC, CT), I32),
        pltpu.VMEM((HPC, 16), F32),
        pltpu.SemaphoreType.DMA,
        pltpu.SemaphoreType.DMA,
    ],
)


def kernel(x, edge_index, batch, W_proj, b_proj, Wq, bq, Wk, bk, Wv, bv,
           Wskip, bskip, Wbeta, gamma, beta_bn):
    src = edge_index[0].astype(I32)
    dst = edge_index[1].astype(I32)
    h = _projin_call(x, W_proj, b_proj.reshape(1, HID))
    ssum = sqsum = None
    for l in range(L):
        wargs = (Wq[l], bq[l].reshape(1, HID), Wk[l], bk[l].reshape(1, HID),
                 Wv[l], bv[l].reshape(1, HID), Wskip[l],
                 bskip[l].reshape(1, HID))
        if l == 0:
            qT4, kvT4, r, bbrow = _proj0_call(h, *wargs)
        else:
            qT4, kvT4, r, bbrow = _projbn_call(
                h, ssum, sqsum, gamma[l - 1].reshape(1, HID),
                beta_bn[l - 1].reshape(1, HID), *wargs)
        bb = jnp.broadcast_to(bbrow.reshape(NSC, HPC, 1), (NSC, HPC, 16))
        num, den = _edge_call(qT4.reshape(NSC * N, HPC * DH),
                              kvT4.reshape(NSC * N, 2 * HPC * DH),
                              src, dst, bb)
        denb = jnp.repeat(
            den.reshape(NSC, HPC, NPAD).transpose(2, 0, 1).reshape(NPAD,
                                                                   HEADS),
            DH, axis=1)
        h, ssum, sqsum = _combine_call(num.reshape(NSC, NPAD, 128), denb, r,
                                       Wbeta[l])
    return _pool_call(h, ssum, sqsum, gamma[L - 1].reshape(1, HID),
                      beta_bn[L - 1].reshape(1, HID),
                      batch.reshape(NBLK, 1, RB).astype(I32))
